# baseline (speedup 1.0000x reference)
import functools

import jax
import jax.numpy as jnp
from jax import lax
from jax.experimental import pallas as pl
from jax.experimental.pallas import tpu as pltpu
from jax.experimental.pallas import tpu_sc as plsc

N = 10000
E = 320000
D_IN = 128
D_H = 128
D_OUT = 64
EPS = 1e-5

NSC = 2
NTILE = 16
NW = NSC * NTILE
EB = 128
RPW = 80
NROWP = NW * RPW
CH = 16
NCH = RPW // CH
NA = N + 8
RA = 624
TAIL = N - NTILE * RA


def _zero_acc(z_hbm, acc, s):
    pltpu.sync_copy(z_hbm.at[pl.ds(s * RA, RA)], acc.at[pl.ds(s * RA, RA)])

    @pl.when(s == NTILE - 1)
    def _zero_tail():
        pltpu.sync_copy(z_hbm.at[pl.ds(NTILE * RA, TAIL)],
                        acc.at[pl.ds(NTILE * RA, TAIL)])


def _copy_out(acc, out_hbm, c, s):
    row0 = c * N + s * RA
    pltpu.sync_copy(acc.at[pl.ds(s * RA, RA)], out_hbm.at[pl.ds(row0, RA)])

    @pl.when(s == NTILE - 1)
    def _copy_tail():
        pltpu.sync_copy(acc.at[pl.ds(NTILE * RA, TAIL)],
                        out_hbm.at[pl.ds(c * N + NTILE * RA, TAIL)])


def _seg_partials(y, src2, dst2, zeros):
    D = y.shape[1]
    mesh = plsc.VectorSubcoreMesh(core_axis_name="c", subcore_axis_name="s")

    @functools.partial(
        pl.kernel, mesh=mesh,
        out_type=jax.ShapeDtypeStruct((2 * N, D), jnp.float32),
        scratch_types=[
            pltpu.VMEM((CH, EB), jnp.int32),
            pltpu.VMEM((CH, EB), jnp.int32),
            pltpu.VMEM((EB, D), jnp.float32),
            pltpu.VMEM((EB, D), jnp.float32),
            pltpu.VMEM_SHARED((NA, D), jnp.float32),
            pltpu.SemaphoreType.DMA,
            pltpu.SemaphoreType.DMA,
            pltpu.SemaphoreType.DMA,
            pltpu.SemaphoreType.DMA,
        ])
    def k(y_hbm, src_hbm, dst_hbm, z_hbm, out_hbm, srcs, dsts,
          rows0, rows1, acc, gsem0, gsem1, ssem0, ssem1):
        c = lax.axis_index("c")
        s = lax.axis_index("s")
        wid = c * NTILE + s
        base_row = wid * RPW

        _zero_acc(z_hbm, acc, s)
        plsc.subcore_barrier()

        rows = (rows0, rows1)
        gsem = (gsem0, gsem1)
        ssem = (ssem0, ssem1)

        def gather(iv, b):
            pltpu.async_copy(y_hbm.at[iv], rows[b % 2], gsem[b % 2])

        def gwait(b):
            pltpu.make_async_copy(y_hbm.at[pl.ds(0, EB)], rows[b % 2],
                                  gsem[b % 2]).wait()

        def sfire(iv, b):
            pltpu.async_copy(rows[b % 2], acc.at[iv], ssem[b % 2], add=True)

        def sdrain(b):
            pltpu.make_async_copy(rows[b % 2], acc.at[pl.ds(0, EB)],
                                  ssem[b % 2]).wait()

        def chunk_run(row0, cnt):
            pltpu.sync_copy(src_hbm.at[pl.ds(row0, cnt)],
                            srcs.at[pl.ds(0, cnt)])
            pltpu.sync_copy(dst_hbm.at[pl.ds(row0, cnt)],
                            dsts.at[pl.ds(0, cnt)])
            gather(srcs.at[0], 0)
            for b in range(cnt):
                gwait(b)
                sfire(dsts.at[b], b)
                if b + 1 < cnt:
                    if b >= 1:
                        sdrain(b - 1)
                    gather(srcs.at[b + 1], b + 1)
            if cnt >= 2:
                sdrain(cnt - 2)
            sdrain(cnt - 1)

        def chunk(cc, carry):
            chunk_run(base_row + cc * CH, CH)
            return carry
        lax.fori_loop(0, NCH, chunk, 0)

        plsc.subcore_barrier()
        _copy_out(acc, out_hbm, c, s)

    return k(y, src2, dst2, zeros)


def _deg_partials(dst2, zeros, ones):
    mesh = plsc.VectorSubcoreMesh(core_axis_name="c", subcore_axis_name="s")

    @functools.partial(
        pl.kernel, mesh=mesh,
        out_type=jax.ShapeDtypeStruct((2 * N, 128), jnp.float32),
        scratch_types=[
            pltpu.VMEM((CH, EB), jnp.int32),
            pltpu.VMEM((EB, 128), jnp.float32),
            pltpu.VMEM_SHARED((NA, 128), jnp.float32),
            pltpu.SemaphoreType.DMA,
        ])
    def k(dst_hbm, z_hbm, ones_hbm, deg_hbm, dsts, onesv, dacc, ssem):
        c = lax.axis_index("c")
        s = lax.axis_index("s")
        wid = c * NTILE + s
        base_row = wid * RPW

        pltpu.sync_copy(ones_hbm, onesv)
        _zero_acc(z_hbm, dacc, s)
        plsc.subcore_barrier()

        def chunk_run(row0, cnt):
            pltpu.sync_copy(dst_hbm.at[pl.ds(row0, cnt)],
                            dsts.at[pl.ds(0, cnt)])
            for b in range(cnt):
                pltpu.async_copy(onesv, dacc.at[dsts.at[b]], ssem, add=True)
            for b in range(cnt):
                pltpu.make_async_copy(onesv, dacc.at[pl.ds(0, EB)],
                                      ssem).wait()

        def chunk(cc, carry):
            chunk_run(base_row + cc * CH, CH)
            return carry
        lax.fori_loop(0, NCH, chunk, 0)

        plsc.subcore_barrier()
        _copy_out(dacc, deg_hbm, c, s)

    return k(dst2, zeros, ones)


_BN = 1000


def _tc_pre(x, Wl, Wr):
    def body(x_ref, wl_ref, wr_ref, y_ref, xr_ref):
        xb = x_ref[...]
        y_ref[...] = jnp.dot(xb, wl_ref[...], preferred_element_type=jnp.float32)
        xr_ref[...] = jnp.dot(xb, wr_ref[...], preferred_element_type=jnp.float32)

    return pl.pallas_call(
        body,
        grid=(N // _BN,),
        in_specs=[
            pl.BlockSpec((_BN, D_IN), lambda i: (i, 0)),
            pl.BlockSpec((D_IN, D_H), lambda i: (0, 0)),
            pl.BlockSpec((D_IN, D_H), lambda i: (0, 0)),
        ],
        out_specs=[pl.BlockSpec((_BN, D_H), lambda i: (i, 0))] * 2,
        out_shape=[jax.ShapeDtypeStruct((N, D_H), jnp.float32)] * 2,
    )(x, Wl, Wr)


def _ln_relu(s, g, be):
    mu = jnp.mean(s, axis=-1, keepdims=True)
    var = jnp.mean((s - mu) ** 2, axis=-1, keepdims=True)
    h = (s - mu) * lax.rsqrt(var + EPS) * g + be
    return jnp.maximum(h, 0.0)


def _tc_mid1(aggp, degp, xr, b, g, be, Wl, Wr):
    D = xr.shape[1]
    Dn = Wl.shape[1]

    def body(agg_ref, dg_ref, xr_ref, b_ref, g_ref, be_ref, wl_ref, wr_ref,
             y_ref, xr2_ref, inv_ref):
        deg = dg_ref[0, :, 0:1] + dg_ref[1, :, 0:1]
        inv = 1.0 / jnp.maximum(deg, 1.0)
        inv_ref[...] = inv
        s = (agg_ref[0] + agg_ref[1]) * inv + xr_ref[...] + b_ref[...]
        h = _ln_relu(s, g_ref[...], be_ref[...])
        y_ref[...] = jnp.dot(h, wl_ref[...], preferred_element_type=jnp.float32)
        xr2_ref[...] = jnp.dot(h, wr_ref[...], preferred_element_type=jnp.float32)

    return pl.pallas_call(
        body,
        grid=(N // _BN,),
        in_specs=[
            pl.BlockSpec((2, _BN, D), lambda i: (0, i, 0)),
            pl.BlockSpec((2, _BN, 128), lambda i: (0, i, 0)),
            pl.BlockSpec((_BN, D), lambda i: (i, 0)),
            pl.BlockSpec((1, D), lambda i: (0, 0)),
            pl.BlockSpec((1, D), lambda i: (0, 0)),
            pl.BlockSpec((1, D), lambda i: (0, 0)),
            pl.BlockSpec((D, Dn), lambda i: (0, 0)),
            pl.BlockSpec((D, Dn), lambda i: (0, 0)),
        ],
        out_specs=[
            pl.BlockSpec((_BN, Dn), lambda i: (i, 0)),
            pl.BlockSpec((_BN, Dn), lambda i: (i, 0)),
            pl.BlockSpec((_BN, 1), lambda i: (i, 0)),
        ],
        out_shape=[
            jax.ShapeDtypeStruct((N, Dn), jnp.float32),
            jax.ShapeDtypeStruct((N, Dn), jnp.float32),
            jax.ShapeDtypeStruct((N, 1), jnp.float32),
        ],
    )(aggp, degp, xr, b, g, be, Wl, Wr)


def _tc_mid2(aggp, inv, xr, b, g, be, Wl, Wr):
    D = xr.shape[1]
    Dn = Wl.shape[1]

    Dy = Wl.shape[1]
    Dx = Wr.shape[1]

    def body(agg_ref, inv_ref, xr_ref, b_ref, g_ref, be_ref, wl_ref, wr_ref,
             y_ref, xr2_ref):
        s = (agg_ref[0] + agg_ref[1]) * inv_ref[...] + xr_ref[...] + b_ref[...]
        h = _ln_relu(s, g_ref[...], be_ref[...])
        y_ref[...] = jnp.dot(h, wl_ref[...], preferred_element_type=jnp.float32)
        xr2_ref[...] = jnp.dot(h, wr_ref[...], preferred_element_type=jnp.float32)

    return pl.pallas_call(
        body,
        grid=(N // _BN,),
        in_specs=[
            pl.BlockSpec((2, _BN, D), lambda i: (0, i, 0)),
            pl.BlockSpec((_BN, 1), lambda i: (i, 0)),
            pl.BlockSpec((_BN, D), lambda i: (i, 0)),
            pl.BlockSpec((1, D), lambda i: (0, 0)),
            pl.BlockSpec((1, D), lambda i: (0, 0)),
            pl.BlockSpec((1, D), lambda i: (0, 0)),
            pl.BlockSpec((D, Dy), lambda i: (0, 0)),
            pl.BlockSpec((D, Dx), lambda i: (0, 0)),
        ],
        out_specs=[
            pl.BlockSpec((_BN, Dy), lambda i: (i, 0)),
            pl.BlockSpec((_BN, Dx), lambda i: (i, 0)),
        ],
        out_shape=[
            jax.ShapeDtypeStruct((N, Dy), jnp.float32),
            jax.ShapeDtypeStruct((N, Dx), jnp.float32),
        ],
    )(aggp, inv, xr, b, g, be, Wl, Wr)


def _tc_fin(aggp, inv, xr, b, g, be, Wc1a, Wc1b, bc1, Wc2p, bc2p):
    D = xr.shape[1]
    G = N // _BN

    def body(agg_ref, inv_ref, xr_ref, b_ref, g_ref, be_ref,
             wa_ref, wb_ref, bc1_ref, w2_ref, bc2_ref, out_ref, sum_s, max_s):
        i = pl.program_id(0)
        agg = agg_ref[0, :, :D] + agg_ref[1, :, :D]
        s = agg * inv_ref[...] + xr_ref[...] + b_ref[...]
        h = _ln_relu(s, g_ref[...], be_ref[...])
        ps = jnp.sum(h, axis=0, keepdims=True)
        pm = jnp.max(h, axis=0, keepdims=True)

        @pl.when(i == 0)
        def _():
            sum_s[...] = ps
            max_s[...] = pm

        @pl.when(i > 0)
        def _():
            sum_s[...] += ps
            max_s[...] = jnp.maximum(max_s[...], pm)

        @pl.when(i == G - 1)
        def _():
            zmean = sum_s[...] * (1.0 / N)
            zmax = max_s[...]
            zz = (jnp.dot(zmean, wa_ref[...], preferred_element_type=jnp.float32)
                  + jnp.dot(zmax, wb_ref[...], preferred_element_type=jnp.float32)
                  + bc1_ref[...])
            zz = jnp.maximum(zz, 0.0)
            out_ref[...] = (jnp.dot(zz, w2_ref[...],
                                    preferred_element_type=jnp.float32)
                            + bc2_ref[...])

    return pl.pallas_call(
        body,
        grid=(G,),
        in_specs=[
            pl.BlockSpec((2, _BN, D_H), lambda i: (0, i, 0)),
            pl.BlockSpec((_BN, 1), lambda i: (i, 0)),
            pl.BlockSpec((_BN, D), lambda i: (i, 0)),
            pl.BlockSpec((1, D), lambda i: (0, 0)),
            pl.BlockSpec((1, D), lambda i: (0, 0)),
            pl.BlockSpec((1, D), lambda i: (0, 0)),
            pl.BlockSpec((D_OUT, D_H), lambda i: (0, 0)),
            pl.BlockSpec((D_OUT, D_H), lambda i: (0, 0)),
            pl.BlockSpec((1, D_H), lambda i: (0, 0)),
            pl.BlockSpec((D_H, 128), lambda i: (0, 0)),
            pl.BlockSpec((1, 128), lambda i: (0, 0)),
        ],
        out_specs=pl.BlockSpec((1, 128), lambda i: (0, 0)),
        out_shape=jax.ShapeDtypeStruct((1, 128), jnp.float32),
        scratch_shapes=[
            pltpu.VMEM((1, D), jnp.float32),
            pltpu.VMEM((1, D), jnp.float32),
        ],
    )(aggp, inv, xr, b, g, be, Wc1a, Wc1b, bc1, Wc2p, bc2p)


def kernel(x, edge_index, Wl1, Wr1, b1, g1, be1, Wl2, Wr2, b2, g2, be2,
           Wl3, Wr3, b3, g3, be3, Wc1, bc1, Wc2, bc2):
    npad = NROWP * EB - E
    src2 = jnp.concatenate(
        [edge_index[0], jnp.zeros((npad,), jnp.int32)]).reshape(NROWP, EB)
    dst2 = jnp.concatenate(
        [edge_index[1], jnp.full((npad,), N, jnp.int32)]).reshape(NROWP, EB)
    row = lambda v: v.reshape(1, -1)

    zeros = jnp.zeros((N, 128), jnp.float32)
    ones = jnp.ones((EB, 128), jnp.float32)
    y1, xr1 = _tc_pre(x, Wl1, Wr1)
    degp = _deg_partials(dst2, zeros, ones)
    agg1 = _seg_partials(y1, src2, dst2, zeros)
    y2, xr2, inv = _tc_mid1(agg1.reshape(2, N, D_H), degp.reshape(2, N, 128),
                            xr1, row(b1), row(g1), row(be1), Wl2, Wr2)

    agg2 = _seg_partials(y2, src2, dst2, zeros)
    Wl3p = jnp.zeros((D_H, 128), jnp.float32).at[:, :D_OUT].set(Wl3)
    y3, xr3 = _tc_mid2(agg2.reshape(2, N, D_H), inv, xr2,
                       row(b2), row(g2), row(be2), Wl3p, Wr3)

    agg3 = _seg_partials(y3, src2, dst2, zeros)
    Wc1a = Wc1[:D_OUT]
    Wc1b = Wc1[D_OUT:]
    Wc2p = jnp.zeros((D_H, 128), jnp.float32).at[:, :Wc2.shape[1]].set(Wc2)
    bc2p = jnp.zeros((1, 128), jnp.float32).at[:, :Wc2.shape[1]].set(bc2)
    outp = _tc_fin(agg3.reshape(2, N, 128), inv, xr3,
                   row(b3), row(g3), row(be3), Wc1a, Wc1b, row(bc1),
                   Wc2p, bc2p)
    return outp[:, :Wc2.shape[1]]


# speedup vs baseline: 2.3233x; 2.3233x over previous
import functools

import jax
import jax.numpy as jnp
from jax import lax
from jax.experimental import pallas as pl
from jax.experimental.pallas import tpu as pltpu
from jax.experimental.pallas import tpu_sc as plsc

N = 10000
E = 320000
D_IN = 128
D_H = 128
D_OUT = 64
EPS = 1e-5

NSC = 2
NTILE = 16
NW = NSC * NTILE
EPW = E // NW
EB = 128
NBF = EPW // EB
ETAIL = EPW - NBF * EB
RA = 624
TAIL = N - NTILE * RA


def _zero_acc(z_hbm, acc, s):
    pltpu.sync_copy(z_hbm.at[pl.ds(s * RA, RA)], acc.at[pl.ds(s * RA, RA)])

    @pl.when(s == NTILE - 1)
    def _zero_tail():
        pltpu.sync_copy(z_hbm.at[pl.ds(NTILE * RA, TAIL)],
                        acc.at[pl.ds(NTILE * RA, TAIL)])


def _copy_out(acc, out_hbm, c, s):
    row0 = c * N + s * RA
    pltpu.sync_copy(acc.at[pl.ds(s * RA, RA)], out_hbm.at[pl.ds(row0, RA)])

    @pl.when(s == NTILE - 1)
    def _copy_tail():
        pltpu.sync_copy(acc.at[pl.ds(NTILE * RA, TAIL)],
                        out_hbm.at[pl.ds(c * N + NTILE * RA, TAIL)])


def _seg_partials(y, src, dst, zeros):
    D = y.shape[1]
    mesh = plsc.VectorSubcoreMesh(core_axis_name="c", subcore_axis_name="s")

    @functools.partial(
        pl.kernel, mesh=mesh,
        out_type=jax.ShapeDtypeStruct((2 * N, D), jnp.float32),
        scratch_types=[
            pltpu.VMEM((EB,), jnp.int32),
            pltpu.VMEM((EB,), jnp.int32),
            pltpu.VMEM((EB,), jnp.int32),
            pltpu.VMEM((EB,), jnp.int32),
            pltpu.VMEM((ETAIL,), jnp.int32),
            pltpu.VMEM((ETAIL,), jnp.int32),
            pltpu.VMEM((EB, D), jnp.float32),
            pltpu.VMEM((EB, D), jnp.float32),
            pltpu.VMEM_SHARED((N, D), jnp.float32),
            pltpu.SemaphoreType.DMA,
            pltpu.SemaphoreType.DMA,
            pltpu.SemaphoreType.DMA,
            pltpu.SemaphoreType.DMA,
        ])
    def k(y_hbm, src_hbm, dst_hbm, z_hbm, out_hbm, src0, dst0, src1, dst1,
          srcT, dstT, rows0, rows1, acc, gsem0, gsem1, ssem0, ssem1):
        c = lax.axis_index("c")
        s = lax.axis_index("s")
        base = (c * NTILE + s) * EPW

        _zero_acc(z_hbm, acc, s)
        plsc.subcore_barrier()

        def load_idx(i, sv, dv):
            off = pl.multiple_of(base + i * EB, 8)
            pltpu.sync_copy(src_hbm.at[pl.ds(off, EB)], sv)
            pltpu.sync_copy(dst_hbm.at[pl.ds(off, EB)], dv)

        def gather(sv, rows, sem):
            pltpu.async_copy(y_hbm.at[sv], rows, sem)

        def gwait(rows, sem):
            pltpu.make_async_copy(y_hbm.at[pl.ds(0, EB)], rows, sem).wait()

        def sfire(rows, dv, sem):
            pltpu.async_copy(rows, acc.at[dv], sem, add=True)

        def sdrain(rows, dv, sem):
            pltpu.make_async_copy(rows, acc.at[dv], sem).wait()

        load_idx(0, src0, dst0)
        gather(src0, rows0, gsem0)
        load_idx(1, src1, dst1)
        gather(src1, rows1, gsem1)

        def pair(kk, carry):
            gwait(rows0, gsem0)
            sfire(rows0, dst0, ssem0)
            gwait(rows1, gsem1)
            sfire(rows1, dst1, ssem1)
            sdrain(rows0, dst0, ssem0)

            @pl.when(2 * kk + 2 < NBF)
            def _():
                load_idx(2 * kk + 2, src0, dst0)
                gather(src0, rows0, gsem0)
            sdrain(rows1, dst1, ssem1)

            @pl.when(2 * kk + 3 < NBF)
            def _():
                load_idx(2 * kk + 3, src1, dst1)
                gather(src1, rows1, gsem1)
            return carry
        lax.fori_loop(0, NBF // 2, pair, 0)

        offT = pl.multiple_of(base + NBF * EB, 8)
        pltpu.sync_copy(src_hbm.at[pl.ds(offT, ETAIL)], srcT)
        pltpu.sync_copy(dst_hbm.at[pl.ds(offT, ETAIL)], dstT)
        pltpu.async_copy(y_hbm.at[srcT], rows0.at[pl.ds(0, ETAIL)],
                         gsem0).wait()
        pltpu.sync_copy(rows0.at[pl.ds(0, ETAIL)], acc.at[dstT], add=True)

        plsc.subcore_barrier()
        _copy_out(acc, out_hbm, c, s)

    return k(y, src, dst, zeros)


def _deg_partials(dst, zeros, ones):
    mesh = plsc.VectorSubcoreMesh(core_axis_name="c", subcore_axis_name="s")

    @functools.partial(
        pl.kernel, mesh=mesh,
        out_type=jax.ShapeDtypeStruct((2 * N, 128), jnp.float32),
        scratch_types=[
            pltpu.VMEM((EB,), jnp.int32),
            pltpu.VMEM((EB,), jnp.int32),
            pltpu.VMEM((ETAIL,), jnp.int32),
            pltpu.VMEM((EB, 128), jnp.float32),
            pltpu.VMEM_SHARED((N, 128), jnp.float32),
            pltpu.SemaphoreType.DMA,
            pltpu.SemaphoreType.DMA,
        ])
    def k(dst_hbm, z_hbm, ones_hbm, deg_hbm, dst0, dst1, dstT, onesv,
          dacc, ssem0, ssem1):
        c = lax.axis_index("c")
        s = lax.axis_index("s")
        base = (c * NTILE + s) * EPW

        pltpu.sync_copy(ones_hbm, onesv)
        _zero_acc(z_hbm, dacc, s)
        plsc.subcore_barrier()

        def load_idx(i, dv):
            off = pl.multiple_of(base + i * EB, 8)
            pltpu.sync_copy(dst_hbm.at[pl.ds(off, EB)], dv)

        def sfire(dv, sem):
            pltpu.async_copy(onesv, dacc.at[dv], sem, add=True)

        def sdrain(dv, sem):
            pltpu.make_async_copy(onesv, dacc.at[dv], sem).wait()

        load_idx(0, dst0)
        load_idx(1, dst1)

        def pair(kk, carry):
            sfire(dst0, ssem0)
            sfire(dst1, ssem1)
            sdrain(dst0, ssem0)

            @pl.when(2 * kk + 2 < NBF)
            def _():
                load_idx(2 * kk + 2, dst0)
            sdrain(dst1, ssem1)

            @pl.when(2 * kk + 3 < NBF)
            def _():
                load_idx(2 * kk + 3, dst1)
            return carry
        lax.fori_loop(0, NBF // 2, pair, 0)

        offT = pl.multiple_of(base + NBF * EB, 8)
        pltpu.sync_copy(dst_hbm.at[pl.ds(offT, ETAIL)], dstT)
        pltpu.sync_copy(onesv.at[pl.ds(0, ETAIL)], dacc.at[dstT], add=True)

        plsc.subcore_barrier()
        _copy_out(dacc, deg_hbm, c, s)

    return k(dst, zeros, ones)


_BN = 1000


def _tc_pre(x, Wl, Wr):
    def body(x_ref, wl_ref, wr_ref, y_ref, xr_ref):
        xb = x_ref[...]
        y_ref[...] = jnp.dot(xb, wl_ref[...], preferred_element_type=jnp.float32)
        xr_ref[...] = jnp.dot(xb, wr_ref[...], preferred_element_type=jnp.float32)

    return pl.pallas_call(
        body,
        grid=(N // _BN,),
        in_specs=[
            pl.BlockSpec((_BN, D_IN), lambda i: (i, 0)),
            pl.BlockSpec((D_IN, D_H), lambda i: (0, 0)),
            pl.BlockSpec((D_IN, D_H), lambda i: (0, 0)),
        ],
        out_specs=[pl.BlockSpec((_BN, D_H), lambda i: (i, 0))] * 2,
        out_shape=[jax.ShapeDtypeStruct((N, D_H), jnp.float32)] * 2,
    )(x, Wl, Wr)


def _ln_relu(s, g, be):
    mu = jnp.mean(s, axis=-1, keepdims=True)
    var = jnp.mean((s - mu) ** 2, axis=-1, keepdims=True)
    h = (s - mu) * lax.rsqrt(var + EPS) * g + be
    return jnp.maximum(h, 0.0)


def _tc_mid1(aggp, degp, xr, b, g, be, Wl, Wr):
    D = xr.shape[1]
    Dn = Wl.shape[1]

    def body(agg_ref, dg_ref, xr_ref, b_ref, g_ref, be_ref, wl_ref, wr_ref,
             y_ref, xr2_ref, inv_ref):
        deg = dg_ref[0, :, 0:1] + dg_ref[1, :, 0:1]
        inv = 1.0 / jnp.maximum(deg, 1.0)
        inv_ref[...] = inv
        s = (agg_ref[0] + agg_ref[1]) * inv + xr_ref[...] + b_ref[...]
        h = _ln_relu(s, g_ref[...], be_ref[...])
        y_ref[...] = jnp.dot(h, wl_ref[...], preferred_element_type=jnp.float32)
        xr2_ref[...] = jnp.dot(h, wr_ref[...], preferred_element_type=jnp.float32)

    return pl.pallas_call(
        body,
        grid=(N // _BN,),
        in_specs=[
            pl.BlockSpec((2, _BN, D), lambda i: (0, i, 0)),
            pl.BlockSpec((2, _BN, 128), lambda i: (0, i, 0)),
            pl.BlockSpec((_BN, D), lambda i: (i, 0)),
            pl.BlockSpec((1, D), lambda i: (0, 0)),
            pl.BlockSpec((1, D), lambda i: (0, 0)),
            pl.BlockSpec((1, D), lambda i: (0, 0)),
            pl.BlockSpec((D, Dn), lambda i: (0, 0)),
            pl.BlockSpec((D, Dn), lambda i: (0, 0)),
        ],
        out_specs=[
            pl.BlockSpec((_BN, Dn), lambda i: (i, 0)),
            pl.BlockSpec((_BN, Dn), lambda i: (i, 0)),
            pl.BlockSpec((_BN, 1), lambda i: (i, 0)),
        ],
        out_shape=[
            jax.ShapeDtypeStruct((N, Dn), jnp.float32),
            jax.ShapeDtypeStruct((N, Dn), jnp.float32),
            jax.ShapeDtypeStruct((N, 1), jnp.float32),
        ],
    )(aggp, degp, xr, b, g, be, Wl, Wr)


def _tc_mid2(aggp, inv, xr, b, g, be, Wl, Wr):
    D = xr.shape[1]
    Dn = Wl.shape[1]

    Dy = Wl.shape[1]
    Dx = Wr.shape[1]

    def body(agg_ref, inv_ref, xr_ref, b_ref, g_ref, be_ref, wl_ref, wr_ref,
             y_ref, xr2_ref):
        s = (agg_ref[0] + agg_ref[1]) * inv_ref[...] + xr_ref[...] + b_ref[...]
        h = _ln_relu(s, g_ref[...], be_ref[...])
        y_ref[...] = jnp.dot(h, wl_ref[...], preferred_element_type=jnp.float32)
        xr2_ref[...] = jnp.dot(h, wr_ref[...], preferred_element_type=jnp.float32)

    return pl.pallas_call(
        body,
        grid=(N // _BN,),
        in_specs=[
            pl.BlockSpec((2, _BN, D), lambda i: (0, i, 0)),
            pl.BlockSpec((_BN, 1), lambda i: (i, 0)),
            pl.BlockSpec((_BN, D), lambda i: (i, 0)),
            pl.BlockSpec((1, D), lambda i: (0, 0)),
            pl.BlockSpec((1, D), lambda i: (0, 0)),
            pl.BlockSpec((1, D), lambda i: (0, 0)),
            pl.BlockSpec((D, Dy), lambda i: (0, 0)),
            pl.BlockSpec((D, Dx), lambda i: (0, 0)),
        ],
        out_specs=[
            pl.BlockSpec((_BN, Dy), lambda i: (i, 0)),
            pl.BlockSpec((_BN, Dx), lambda i: (i, 0)),
        ],
        out_shape=[
            jax.ShapeDtypeStruct((N, Dy), jnp.float32),
            jax.ShapeDtypeStruct((N, Dx), jnp.float32),
        ],
    )(aggp, inv, xr, b, g, be, Wl, Wr)


def _tc_fin(aggp, inv, xr, b, g, be, Wc1a, Wc1b, bc1, Wc2p, bc2p):
    D = xr.shape[1]
    G = N // _BN

    def body(agg_ref, inv_ref, xr_ref, b_ref, g_ref, be_ref,
             wa_ref, wb_ref, bc1_ref, w2_ref, bc2_ref, out_ref, sum_s, max_s):
        i = pl.program_id(0)
        agg = agg_ref[0, :, :D] + agg_ref[1, :, :D]
        s = agg * inv_ref[...] + xr_ref[...] + b_ref[...]
        h = _ln_relu(s, g_ref[...], be_ref[...])
        ps = jnp.sum(h, axis=0, keepdims=True)
        pm = jnp.max(h, axis=0, keepdims=True)

        @pl.when(i == 0)
        def _():
            sum_s[...] = ps
            max_s[...] = pm

        @pl.when(i > 0)
        def _():
            sum_s[...] += ps
            max_s[...] = jnp.maximum(max_s[...], pm)

        @pl.when(i == G - 1)
        def _():
            zmean = sum_s[...] * (1.0 / N)
            zmax = max_s[...]
            zz = (jnp.dot(zmean, wa_ref[...], preferred_element_type=jnp.float32)
                  + jnp.dot(zmax, wb_ref[...], preferred_element_type=jnp.float32)
                  + bc1_ref[...])
            zz = jnp.maximum(zz, 0.0)
            out_ref[...] = (jnp.dot(zz, w2_ref[...],
                                    preferred_element_type=jnp.float32)
                            + bc2_ref[...])

    return pl.pallas_call(
        body,
        grid=(G,),
        in_specs=[
            pl.BlockSpec((2, _BN, D_H), lambda i: (0, i, 0)),
            pl.BlockSpec((_BN, 1), lambda i: (i, 0)),
            pl.BlockSpec((_BN, D), lambda i: (i, 0)),
            pl.BlockSpec((1, D), lambda i: (0, 0)),
            pl.BlockSpec((1, D), lambda i: (0, 0)),
            pl.BlockSpec((1, D), lambda i: (0, 0)),
            pl.BlockSpec((D_OUT, D_H), lambda i: (0, 0)),
            pl.BlockSpec((D_OUT, D_H), lambda i: (0, 0)),
            pl.BlockSpec((1, D_H), lambda i: (0, 0)),
            pl.BlockSpec((D_H, 128), lambda i: (0, 0)),
            pl.BlockSpec((1, 128), lambda i: (0, 0)),
        ],
        out_specs=pl.BlockSpec((1, 128), lambda i: (0, 0)),
        out_shape=jax.ShapeDtypeStruct((1, 128), jnp.float32),
        scratch_shapes=[
            pltpu.VMEM((1, D), jnp.float32),
            pltpu.VMEM((1, D), jnp.float32),
        ],
    )(aggp, inv, xr, b, g, be, Wc1a, Wc1b, bc1, Wc2p, bc2p)


def kernel(x, edge_index, Wl1, Wr1, b1, g1, be1, Wl2, Wr2, b2, g2, be2,
           Wl3, Wr3, b3, g3, be3, Wc1, bc1, Wc2, bc2):
    src = edge_index[0]
    dst = edge_index[1]
    row = lambda v: v.reshape(1, -1)

    zeros = jnp.zeros((N, 128), jnp.float32)
    ones = jnp.ones((EB, 128), jnp.float32)
    y1, xr1 = _tc_pre(x, Wl1, Wr1)
    degp = _deg_partials(dst, zeros, ones)
    agg1 = _seg_partials(y1, src, dst, zeros)
    y2, xr2, inv = _tc_mid1(agg1.reshape(2, N, D_H), degp.reshape(2, N, 128),
                            xr1, row(b1), row(g1), row(be1), Wl2, Wr2)

    agg2 = _seg_partials(y2, src, dst, zeros)
    Wl3p = jnp.zeros((D_H, 128), jnp.float32).at[:, :D_OUT].set(Wl3)
    y3, xr3 = _tc_mid2(agg2.reshape(2, N, D_H), inv, xr2,
                       row(b2), row(g2), row(be2), Wl3p, Wr3)

    agg3 = _seg_partials(y3, src, dst, zeros)
    Wc1a = Wc1[:D_OUT]
    Wc1b = Wc1[D_OUT:]
    Wc2p = jnp.zeros((D_H, 128), jnp.float32).at[:, :Wc2.shape[1]].set(Wc2)
    bc2p = jnp.zeros((1, 128), jnp.float32).at[:, :Wc2.shape[1]].set(bc2)
    outp = _tc_fin(agg3.reshape(2, N, 128), inv, xr3,
                   row(b3), row(g3), row(be3), Wc1a, Wc1b, row(bc1),
                   Wc2p, bc2p)
    return outp[:, :Wc2.shape[1]]


# speedup vs baseline: 2.3904x; 1.0289x over previous
import functools

import jax
import jax.numpy as jnp
from jax import lax
from jax.experimental import pallas as pl
from jax.experimental.pallas import tpu as pltpu
from jax.experimental.pallas import tpu_sc as plsc

N = 10000
E = 320000
D_IN = 128
D_H = 128
D_OUT = 64
EPS = 1e-5

NSC = 2
NTILE = 16
NW = NSC * NTILE
EPW = E // NW
EB = 128
NBF = EPW // EB
ETAIL = EPW - NBF * EB
RA = 624
TAIL = N - NTILE * RA


def _zero_acc(z_hbm, acc, s):
    pltpu.sync_copy(z_hbm.at[pl.ds(s * RA, RA)], acc.at[pl.ds(s * RA, RA)])

    @pl.when(s == NTILE - 1)
    def _zero_tail():
        pltpu.sync_copy(z_hbm.at[pl.ds(NTILE * RA, TAIL)],
                        acc.at[pl.ds(NTILE * RA, TAIL)])


def _copy_out(acc, out_hbm, c, s):
    row0 = c * N + s * RA
    pltpu.sync_copy(acc.at[pl.ds(s * RA, RA)], out_hbm.at[pl.ds(row0, RA)])

    @pl.when(s == NTILE - 1)
    def _copy_tail():
        pltpu.sync_copy(acc.at[pl.ds(NTILE * RA, TAIL)],
                        out_hbm.at[pl.ds(c * N + NTILE * RA, TAIL)])


def _seg_partials(y, src, dst, zeros):
    D = y.shape[1]
    mesh = plsc.VectorSubcoreMesh(core_axis_name="c", subcore_axis_name="s")

    @functools.partial(
        pl.kernel, mesh=mesh,
        out_type=jax.ShapeDtypeStruct((2 * N, D), jnp.float32),
        scratch_types=[
            pltpu.VMEM((EB,), jnp.int32),
            pltpu.VMEM((EB,), jnp.int32),
            pltpu.VMEM((EB,), jnp.int32),
            pltpu.VMEM((EB,), jnp.int32),
            pltpu.VMEM((ETAIL,), jnp.int32),
            pltpu.VMEM((ETAIL,), jnp.int32),
            pltpu.VMEM((EB, D), jnp.float32),
            pltpu.VMEM((EB, D), jnp.float32),
            pltpu.VMEM_SHARED((N, D), jnp.float32),
            pltpu.SemaphoreType.DMA,
            pltpu.SemaphoreType.DMA,
            pltpu.SemaphoreType.DMA,
            pltpu.SemaphoreType.DMA,
        ])
    def k(y_hbm, src_hbm, dst_hbm, z_hbm, out_hbm, src0, dst0, src1, dst1,
          srcT, dstT, rows0, rows1, acc, gsem0, gsem1, ssem0, ssem1):
        c = lax.axis_index("c")
        s = lax.axis_index("s")
        base = (c * NTILE + s) * EPW

        _zero_acc(z_hbm, acc, s)
        plsc.subcore_barrier()

        def load_idx(i, sv, dv):
            off = pl.multiple_of(base + i * EB, 8)
            pltpu.sync_copy(src_hbm.at[pl.ds(off, EB)], sv)
            pltpu.sync_copy(dst_hbm.at[pl.ds(off, EB)], dv)

        def gather(sv, rows, sem):
            pltpu.async_copy(y_hbm.at[sv], rows, sem)

        def gwait(rows, sem):
            pltpu.make_async_copy(y_hbm.at[pl.ds(0, EB)], rows, sem).wait()

        def sfire(rows, dv, sem):
            pltpu.async_copy(rows, acc.at[dv], sem, add=True)

        def sdrain(rows, dv, sem):
            pltpu.make_async_copy(rows, acc.at[dv], sem).wait()

        load_idx(0, src0, dst0)
        gather(src0, rows0, gsem0)
        load_idx(1, src1, dst1)
        gather(src1, rows1, gsem1)

        def pair(kk, carry):
            gwait(rows0, gsem0)
            sfire(rows0, dst0, ssem0)
            gwait(rows1, gsem1)
            sfire(rows1, dst1, ssem1)
            sdrain(rows0, dst0, ssem0)

            @pl.when(2 * kk + 2 < NBF)
            def _():
                load_idx(2 * kk + 2, src0, dst0)
                gather(src0, rows0, gsem0)
            sdrain(rows1, dst1, ssem1)

            @pl.when(2 * kk + 3 < NBF)
            def _():
                load_idx(2 * kk + 3, src1, dst1)
                gather(src1, rows1, gsem1)
            return carry
        lax.fori_loop(0, NBF // 2, pair, 0)

        offT = pl.multiple_of(base + NBF * EB, 8)
        pltpu.sync_copy(src_hbm.at[pl.ds(offT, ETAIL)], srcT)
        pltpu.sync_copy(dst_hbm.at[pl.ds(offT, ETAIL)], dstT)
        pltpu.async_copy(y_hbm.at[srcT], rows0.at[pl.ds(0, ETAIL)],
                         gsem0).wait()
        pltpu.sync_copy(rows0.at[pl.ds(0, ETAIL)], acc.at[dstT], add=True)

        plsc.subcore_barrier()
        _copy_out(acc, out_hbm, c, s)

    return k(y, src, dst, zeros)


def _deg_partials(dst, zeros, ones):
    mesh = plsc.VectorSubcoreMesh(core_axis_name="c", subcore_axis_name="s")

    @functools.partial(
        pl.kernel, mesh=mesh,
        out_type=jax.ShapeDtypeStruct((2 * N, 128), jnp.float32),
        scratch_types=[
            pltpu.VMEM((EB,), jnp.int32),
            pltpu.VMEM((EB,), jnp.int32),
            pltpu.VMEM((ETAIL,), jnp.int32),
            pltpu.VMEM((EB, 128), jnp.float32),
            pltpu.VMEM_SHARED((N, 128), jnp.float32),
            pltpu.SemaphoreType.DMA,
            pltpu.SemaphoreType.DMA,
        ])
    def k(dst_hbm, z_hbm, ones_hbm, deg_hbm, dst0, dst1, dstT, onesv,
          dacc, ssem0, ssem1):
        c = lax.axis_index("c")
        s = lax.axis_index("s")
        base = (c * NTILE + s) * EPW

        pltpu.sync_copy(ones_hbm, onesv)
        _zero_acc(z_hbm, dacc, s)
        plsc.subcore_barrier()

        def load_idx(i, dv):
            off = pl.multiple_of(base + i * EB, 8)
            pltpu.sync_copy(dst_hbm.at[pl.ds(off, EB)], dv)

        def sfire(dv, sem):
            pltpu.async_copy(onesv, dacc.at[dv], sem, add=True)

        def sdrain(dv, sem):
            pltpu.make_async_copy(onesv, dacc.at[dv], sem).wait()

        load_idx(0, dst0)
        load_idx(1, dst1)

        def pair(kk, carry):
            sfire(dst0, ssem0)
            sfire(dst1, ssem1)
            sdrain(dst0, ssem0)

            @pl.when(2 * kk + 2 < NBF)
            def _():
                load_idx(2 * kk + 2, dst0)
            sdrain(dst1, ssem1)

            @pl.when(2 * kk + 3 < NBF)
            def _():
                load_idx(2 * kk + 3, dst1)
            return carry
        lax.fori_loop(0, NBF // 2, pair, 0)

        offT = pl.multiple_of(base + NBF * EB, 8)
        pltpu.sync_copy(dst_hbm.at[pl.ds(offT, ETAIL)], dstT)
        pltpu.sync_copy(onesv.at[pl.ds(0, ETAIL)], dacc.at[dstT], add=True)

        plsc.subcore_barrier()
        _copy_out(dacc, deg_hbm, c, s)

    return k(dst, zeros, ones)


_BN = 1000


def _ln_relu(s, g, be):
    mu = jnp.mean(s, axis=-1, keepdims=True)
    var = jnp.mean((s - mu) ** 2, axis=-1, keepdims=True)
    h = (s - mu) / jnp.sqrt(var + EPS) * g + be
    return jnp.maximum(h, 0.0)


def _tc_layer1(aggp, degp, x, b, g, be, Wl, Wr):
    def body(agg_ref, dg_ref, x_ref, b_ref, g_ref, be_ref, wl_ref, wr_ref,
             h_ref, degc_ref):
        deg = dg_ref[0, :, 0:1] + dg_ref[1, :, 0:1]
        degc = jnp.maximum(deg, 1.0)
        degc_ref[...] = degc
        agg = (agg_ref[0] + agg_ref[1]) / degc
        s = (jnp.dot(agg, wl_ref[...], preferred_element_type=jnp.float32)
             + jnp.dot(x_ref[...], wr_ref[...],
                       preferred_element_type=jnp.float32) + b_ref[...])
        h_ref[...] = _ln_relu(s, g_ref[...], be_ref[...])

    return pl.pallas_call(
        body,
        grid=(N // _BN,),
        in_specs=[
            pl.BlockSpec((2, _BN, D_IN), lambda i: (0, i, 0)),
            pl.BlockSpec((2, _BN, 128), lambda i: (0, i, 0)),
            pl.BlockSpec((_BN, D_IN), lambda i: (i, 0)),
            pl.BlockSpec((1, D_H), lambda i: (0, 0)),
            pl.BlockSpec((1, D_H), lambda i: (0, 0)),
            pl.BlockSpec((1, D_H), lambda i: (0, 0)),
            pl.BlockSpec((D_IN, D_H), lambda i: (0, 0)),
            pl.BlockSpec((D_IN, D_H), lambda i: (0, 0)),
        ],
        out_specs=[
            pl.BlockSpec((_BN, D_H), lambda i: (i, 0)),
            pl.BlockSpec((_BN, 1), lambda i: (i, 0)),
        ],
        out_shape=[
            jax.ShapeDtypeStruct((N, D_H), jnp.float32),
            jax.ShapeDtypeStruct((N, 1), jnp.float32),
        ],
    )(aggp, degp, x, b, g, be, Wl, Wr)


def _tc_layer2(aggp, degc, hprev, b, g, be, Wl, Wr):
    def body(agg_ref, dc_ref, h_ref, b_ref, g_ref, be_ref, wl_ref, wr_ref,
             out_ref):
        agg = (agg_ref[0] + agg_ref[1]) / dc_ref[...]
        s = (jnp.dot(agg, wl_ref[...], preferred_element_type=jnp.float32)
             + jnp.dot(h_ref[...], wr_ref[...],
                       preferred_element_type=jnp.float32) + b_ref[...])
        out_ref[...] = _ln_relu(s, g_ref[...], be_ref[...])

    return pl.pallas_call(
        body,
        grid=(N // _BN,),
        in_specs=[
            pl.BlockSpec((2, _BN, D_H), lambda i: (0, i, 0)),
            pl.BlockSpec((_BN, 1), lambda i: (i, 0)),
            pl.BlockSpec((_BN, D_H), lambda i: (i, 0)),
            pl.BlockSpec((1, D_H), lambda i: (0, 0)),
            pl.BlockSpec((1, D_H), lambda i: (0, 0)),
            pl.BlockSpec((1, D_H), lambda i: (0, 0)),
            pl.BlockSpec((D_H, D_H), lambda i: (0, 0)),
            pl.BlockSpec((D_H, D_H), lambda i: (0, 0)),
        ],
        out_specs=pl.BlockSpec((_BN, D_H), lambda i: (i, 0)),
        out_shape=jax.ShapeDtypeStruct((N, D_H), jnp.float32),
    )(aggp, degc, hprev, b, g, be, Wl, Wr)


def _tc_fin(aggp, degc, hprev, b, g, be, Wl, Wr, Wc1a, Wc1b, bc1, Wc2p,
            bc2p):
    G = N // _BN

    def body(agg_ref, dc_ref, h_ref, b_ref, g_ref, be_ref, wl_ref, wr_ref,
             wa_ref, wb_ref, bc1_ref, w2_ref, bc2_ref, out_ref, sum_s,
             max_s):
        i = pl.program_id(0)
        agg = (agg_ref[0] + agg_ref[1]) / dc_ref[...]
        s = (jnp.dot(agg, wl_ref[...], preferred_element_type=jnp.float32)
             + jnp.dot(h_ref[...], wr_ref[...],
                       preferred_element_type=jnp.float32) + b_ref[...])
        h = _ln_relu(s, g_ref[...], be_ref[...])
        ps = jnp.sum(h, axis=0, keepdims=True)
        pm = jnp.max(h, axis=0, keepdims=True)

        @pl.when(i == 0)
        def _():
            sum_s[...] = ps
            max_s[...] = pm

        @pl.when(i > 0)
        def _():
            sum_s[...] += ps
            max_s[...] = jnp.maximum(max_s[...], pm)

        @pl.when(i == G - 1)
        def _():
            zmean = sum_s[...] * (1.0 / N)
            zmax = max_s[...]
            zz = (jnp.dot(zmean, wa_ref[...],
                          preferred_element_type=jnp.float32)
                  + jnp.dot(zmax, wb_ref[...],
                            preferred_element_type=jnp.float32)
                  + bc1_ref[...])
            zz = jnp.maximum(zz, 0.0)
            out_ref[...] = (jnp.dot(zz, w2_ref[...],
                                    preferred_element_type=jnp.float32)
                            + bc2_ref[...])

    return pl.pallas_call(
        body,
        grid=(G,),
        in_specs=[
            pl.BlockSpec((2, _BN, D_H), lambda i: (0, i, 0)),
            pl.BlockSpec((_BN, 1), lambda i: (i, 0)),
            pl.BlockSpec((_BN, D_H), lambda i: (i, 0)),
            pl.BlockSpec((1, D_OUT), lambda i: (0, 0)),
            pl.BlockSpec((1, D_OUT), lambda i: (0, 0)),
            pl.BlockSpec((1, D_OUT), lambda i: (0, 0)),
            pl.BlockSpec((D_H, D_OUT), lambda i: (0, 0)),
            pl.BlockSpec((D_H, D_OUT), lambda i: (0, 0)),
            pl.BlockSpec((D_OUT, D_H), lambda i: (0, 0)),
            pl.BlockSpec((D_OUT, D_H), lambda i: (0, 0)),
            pl.BlockSpec((1, D_H), lambda i: (0, 0)),
            pl.BlockSpec((D_H, 128), lambda i: (0, 0)),
            pl.BlockSpec((1, 128), lambda i: (0, 0)),
        ],
        out_specs=pl.BlockSpec((1, 128), lambda i: (0, 0)),
        out_shape=jax.ShapeDtypeStruct((1, 128), jnp.float32),
        scratch_shapes=[
            pltpu.VMEM((1, D_OUT), jnp.float32),
            pltpu.VMEM((1, D_OUT), jnp.float32),
        ],
    )(aggp, degc, hprev, b, g, be, Wl, Wr, Wc1a, Wc1b, bc1, Wc2p, bc2p)


def kernel(x, edge_index, Wl1, Wr1, b1, g1, be1, Wl2, Wr2, b2, g2, be2,
           Wl3, Wr3, b3, g3, be3, Wc1, bc1, Wc2, bc2):
    src = edge_index[0]
    dst = edge_index[1]
    row = lambda v: v.reshape(1, -1)
    zeros = jnp.zeros((N, 128), jnp.float32)
    ones = jnp.ones((EB, 128), jnp.float32)

    degp = _deg_partials(dst, zeros, ones)
    agg1 = _seg_partials(x, src, dst, zeros)
    h1, degc = _tc_layer1(agg1.reshape(2, N, D_IN), degp.reshape(2, N, 128),
                          x, row(b1), row(g1), row(be1), Wl1, Wr1)

    agg2 = _seg_partials(h1, src, dst, zeros)
    h2 = _tc_layer2(agg2.reshape(2, N, D_H), degc, h1,
                    row(b2), row(g2), row(be2), Wl2, Wr2)

    agg3 = _seg_partials(h2, src, dst, zeros)
    Wc1a = Wc1[:D_OUT]
    Wc1b = Wc1[D_OUT:]
    Wc2p = jnp.zeros((D_H, 128), jnp.float32).at[:, :Wc2.shape[1]].set(Wc2)
    bc2p = jnp.zeros((1, 128), jnp.float32).at[:, :Wc2.shape[1]].set(bc2)
    outp = _tc_fin(agg3.reshape(2, N, D_H), degc, h2,
                   row(b3), row(g3), row(be3), Wl3, Wr3,
                   Wc1a, Wc1b, row(bc1), Wc2p, bc2p)
    return outp[:, :Wc2.shape[1]]
